# baseline (speedup 1.0000x reference)
import functools

import jax
import jax.numpy as jnp
from jax import lax
from jax.experimental import pallas as pl
from jax.experimental.pallas import tpu as pltpu
from jax.experimental.pallas import tpu_sc as plsc

SEQ = 200
DIM = 32
BATCH = 4096
VOCAB = 1000000

_NC = 2
_NS = 16
_NW = _NC * _NS

B_PER_W = BATCH // _NW
_SR = SEQ // 8
_NBUF = 4


def _emb_body(idx6_hbm, table_hbm, pos_hbm, out_hbm,
              idxT, rows4, outbuf, pos_v, scnt, isem, gsem, osem):
    w = lax.axis_index("s") * _NC + lax.axis_index("c")

    pltpu.sync_copy(pos_hbm, pos_v)

    stages = []
    for sr in range(_SR):
        stages.append(pltpu.async_copy(
            idx6_hbm.at[pl.ds((sr * 32 + w) * 8, 8)],
            idxT.at[pl.ds(sr * 8, 8)],
            isem,
        ))
    for cp in stages:
        cp.wait()

    for s0 in range(_NBUF - 1):
        pltpu.async_copy(
            table_hbm.at[idxT.at[s0]],
            rows4.at[pl.ds(s0 * B_PER_W, B_PER_W)],
            gsem.at[s0],
        )

    iota = lax.iota(jnp.int32, 16)
    row_idx = [
        [iota + (k * B_PER_W + b0) for b0 in range(0, B_PER_W, 16)]
        for k in range(_NBUF)
    ]
    col_idx = [jnp.full((16,), d, jnp.int32) for d in range(DIM)]
    scnt[pl.ds(0, 16)] = iota * 0

    def group_body(s4, carry):
        s_base = s4 * _NBUF
        svec = scnt[pl.ds(0, 16)]
        for k in range(_NBUF):
            s = s_base + k
            q = k & 1

            pltpu.make_async_copy(
                table_hbm.at[idxT.at[s]],
                rows4.at[pl.ds(k * B_PER_W, B_PER_W)],
                gsem.at[k],
            ).wait()

            @pl.when(s + _NBUF - 1 < SEQ)
            def _():
                kn = (k + _NBUF - 1) % _NBUF
                pltpu.async_copy(
                    table_hbm.at[idxT.at[s + _NBUF - 1]],
                    rows4.at[pl.ds(kn * B_PER_W, B_PER_W)],
                    gsem.at[kn],
                )

            @pl.when(s >= 2)
            def _():
                for dt in range(4):
                    pltpu.make_async_copy(
                        outbuf.at[pl.ds(q * 32 + dt * 8, 8)],
                        out_hbm.at[
                            pl.ds((((s - 2) * 4 + dt) * 32 + w) * 8, 8)
                        ],
                        osem.at[q],
                    ).wait()

            sk = svec + k
            for d in range(DIM):
                pv = plsc.load_gather(pos_v, [sk, col_idx[d]])
                for g in range(B_PER_W // 16):
                    val = plsc.load_gather(
                        rows4, [row_idx[k][g], col_idx[d]]
                    )
                    outbuf[q * 32 + (d // 8) * 8 + d % 8, pl.ds(g * 16, 16)] = (
                        val + pv
                    )

            for dt in range(4):
                pltpu.async_copy(
                    outbuf.at[pl.ds(q * 32 + dt * 8, 8)],
                    out_hbm.at[pl.ds(((s * 4 + dt) * 32 + w) * 8, 8)],
                    osem.at[q],
                )
        scnt[pl.ds(0, 16)] = svec + _NBUF
        return carry

    lax.fori_loop(0, SEQ // _NBUF, group_body, 0)

    for sd in (SEQ - 2, SEQ - 1):
        q = sd & 1
        for dt in range(4):
            pltpu.make_async_copy(
                outbuf.at[pl.ds(q * 32 + dt * 8, 8)],
                out_hbm.at[pl.ds(((sd * 4 + dt) * 32 + w) * 8, 8)],
                osem.at[q],
            ).wait()


_emb = functools.partial(
    pl.kernel,
    mesh=plsc.VectorSubcoreMesh(core_axis_name="c", subcore_axis_name="s"),
    out_type=jax.ShapeDtypeStruct((BATCH * SEQ * DIM // 128, 128), jnp.float32),
    scratch_types=[
        pltpu.VMEM((SEQ, B_PER_W), jnp.int32),
        pltpu.VMEM((_NBUF * B_PER_W, DIM), jnp.float32),
        pltpu.VMEM((2 * 4 * 8, 128), jnp.float32),
        pltpu.VMEM((SEQ, DIM), jnp.float32),
        pltpu.VMEM((16,), jnp.int32),
        pltpu.SemaphoreType.DMA,
        pltpu.SemaphoreType.DMA((_NBUF,)),
        pltpu.SemaphoreType.DMA((2,)),
    ],
    compiler_params=pltpu.CompilerParams(
        use_tc_tiling_on_sc=False, needs_layout_passes=False
    ),
)(_emb_body)


def kernel(inputs, word_table, pos_table):
    scaled = inputs.astype(jnp.int32) * 4
    idx6 = (
        scaled.T.reshape(_SR, 8, 32, 128).transpose(0, 2, 1, 3).reshape(-1, 128)
    )
    table_padded = jnp.pad(word_table, ((0, 0), (0, 128 - DIM))).reshape(
        4 * VOCAB, DIM
    )
    out = _emb(idx6, table_padded, pos_table)
    outr = out.reshape(SEQ, DIM // 8, BATCH // 128, 8, 128)
    return outr.transpose(2, 4, 0, 1, 3).reshape(BATCH, SEQ, DIM)


# speedup vs baseline: 1.6715x; 1.6715x over previous
import functools

import jax
import jax.numpy as jnp
from jax import lax
from jax.experimental import pallas as pl
from jax.experimental.pallas import tpu as pltpu
from jax.experimental.pallas import tpu_sc as plsc

SEQ = 200
DIM = 32
BATCH = 4096
VOCAB = 1000000

_NC = 2
_NS = 16
_NW = _NC * _NS

ROWS_PER_W = (BATCH * SEQ) // _NW
CHUNK_BATCH = 4
CHUNK = CHUNK_BATCH * SEQ
N_CHUNKS = ROWS_PER_W // CHUNK


def _emb_body(idx_hbm, table_hbm, pos_hbm, out_hbm,
              idx_v, rows_v, pos_v, gsem, osem):
    wid = lax.axis_index("s") * _NC + lax.axis_index("c")
    base = wid * ROWS_PER_W

    pltpu.sync_copy(pos_hbm, pos_v)
    pltpu.sync_copy(idx_hbm.at[pl.ds(base, ROWS_PER_W)], idx_v)

    def gather(g, k):
        return pltpu.async_copy(
            table_hbm.at[idx_v.at[pl.ds(g * CHUNK, CHUNK)]],
            rows_v.at[pl.ds(k * CHUNK, CHUNK)],
            gsem.at[k],
        )

    def write(g, k):
        return pltpu.async_copy(
            rows_v.at[pl.ds(k * CHUNK, CHUNK)],
            out_hbm.at[pl.ds(base + g * CHUNK, CHUNK), pl.ds(0, DIM)],
            osem.at[k],
        )

    def wait_gather(g, k):
        pltpu.make_async_copy(
            table_hbm.at[idx_v.at[pl.ds(g * CHUNK, CHUNK)]],
            rows_v.at[pl.ds(k * CHUNK, CHUNK)],
            gsem.at[k],
        ).wait()

    def wait_write(g, k):
        pltpu.make_async_copy(
            rows_v.at[pl.ds(k * CHUNK, CHUNK)],
            out_hbm.at[pl.ds(base + g * CHUNK, CHUNK), pl.ds(0, DIM)],
            osem.at[k],
        ).wait()

    gather(0, 0)

    def g2_body(g2, carry):
        g0 = g2 * 2
        for k in range(2):
            g = g0 + k
            other = 1 - k
            wait_gather(g, k)

            @pl.when(g >= 1)
            def _():
                wait_write(g - 1, other)

            @pl.when(g + 1 < N_CHUNKS)
            def _():
                gather(g + 1, other)

            def s_body(s, c):
                p0 = pos_v[s, pl.ds(0, 16)]
                p1 = pos_v[s, pl.ds(16, 16)]
                for r in range(CHUNK_BATCH):
                    q = k * CHUNK + r * SEQ + s
                    rows_v[q, pl.ds(0, 16)] = rows_v[q, pl.ds(0, 16)] + p0
                    rows_v[q, pl.ds(16, 16)] = rows_v[q, pl.ds(16, 16)] + p1
                return c

            lax.fori_loop(0, SEQ, s_body, 0)
            write(g, k)
        return carry

    lax.fori_loop(0, N_CHUNKS // 2, g2_body, 0)
    wait_write(N_CHUNKS - 1, 1)


_emb = functools.partial(
    pl.kernel,
    mesh=plsc.VectorSubcoreMesh(core_axis_name="c", subcore_axis_name="s"),
    out_type=jax.ShapeDtypeStruct((BATCH * SEQ, 128), jnp.float32),
    scratch_types=[
        pltpu.VMEM((ROWS_PER_W,), jnp.int32),
        pltpu.VMEM((2 * CHUNK, DIM), jnp.float32),
        pltpu.VMEM((SEQ, DIM), jnp.float32),
        pltpu.SemaphoreType.DMA((2,)),
        pltpu.SemaphoreType.DMA((2,)),
    ],
    compiler_params=pltpu.CompilerParams(use_tc_tiling_on_sc=False),
)(_emb_body)


def kernel(inputs, word_table, pos_table):
    flat_idx = inputs.reshape(-1).astype(jnp.int32) * 4
    table_padded = jnp.pad(word_table, ((0, 0), (0, 128 - DIM))).reshape(
        4 * VOCAB, DIM
    )
    out = _emb(flat_idx, table_padded, pos_table)
    return out[:, :DIM].reshape(inputs.shape[0], inputs.shape[1], DIM)


# speedup vs baseline: 1.6724x; 1.0005x over previous
import functools

import jax
import jax.numpy as jnp
from jax import lax
from jax.experimental import pallas as pl
from jax.experimental.pallas import tpu as pltpu
from jax.experimental.pallas import tpu_sc as plsc

SEQ = 200
DIM = 32
BATCH = 4096
VOCAB = 1000000

_NC = 2
_NS = 16
_NW = _NC * _NS

ROWS_PER_W = (BATCH * SEQ) // _NW
CHUNK_BATCH = 4
CHUNK = CHUNK_BATCH * SEQ
N_CHUNKS = ROWS_PER_W // CHUNK


def _emb_body(idx_hbm, table_hbm, pos_hbm, out_hbm,
              idx_v, rows_v, pos_v, gsem, osem):
    wid = lax.axis_index("s") * _NC + lax.axis_index("c")
    base = wid * ROWS_PER_W

    pltpu.sync_copy(pos_hbm, pos_v)
    pltpu.sync_copy(idx_hbm.at[pl.ds(base, ROWS_PER_W)], idx_v)

    def gather(g, k):
        return pltpu.async_copy(
            table_hbm.at[idx_v.at[pl.ds(g * CHUNK, CHUNK)]],
            rows_v.at[pl.ds(k * CHUNK, CHUNK)],
            gsem.at[k],
        )

    def write(g, k):
        return pltpu.async_copy(
            rows_v.at[pl.ds(k * CHUNK, CHUNK)],
            out_hbm.at[pl.ds(base + g * CHUNK, CHUNK), pl.ds(0, DIM)],
            osem.at[k],
        )

    def wait_gather(g, k):
        pltpu.make_async_copy(
            table_hbm.at[idx_v.at[pl.ds(g * CHUNK, CHUNK)]],
            rows_v.at[pl.ds(k * CHUNK, CHUNK)],
            gsem.at[k],
        ).wait()

    def wait_write(g, k):
        pltpu.make_async_copy(
            rows_v.at[pl.ds(k * CHUNK, CHUNK)],
            out_hbm.at[pl.ds(base + g * CHUNK, CHUNK), pl.ds(0, DIM)],
            osem.at[k],
        ).wait()

    gather(0, 0)

    def g2_body(g2, carry):
        g0 = g2 * 2
        for k in range(2):
            g = g0 + k
            other = 1 - k
            wait_gather(g, k)

            @pl.when(g >= 1)
            def _():
                wait_write(g - 1, other)

            @pl.when(g + 1 < N_CHUNKS)
            def _():
                gather(g + 1, other)

            def s_body(s, c):
                p0 = pos_v[s, pl.ds(0, 16)]
                p1 = pos_v[s, pl.ds(16, 16)]
                for r in range(CHUNK_BATCH):
                    q = k * CHUNK + r * SEQ + s
                    rows_v[q, pl.ds(0, 16)] = rows_v[q, pl.ds(0, 16)] + p0
                    rows_v[q, pl.ds(16, 16)] = rows_v[q, pl.ds(16, 16)] + p1
                return c

            lax.fori_loop(0, SEQ, s_body, 0, unroll=4)
            write(g, k)
        return carry

    lax.fori_loop(0, N_CHUNKS // 2, g2_body, 0)
    wait_write(N_CHUNKS - 1, 1)


_emb = functools.partial(
    pl.kernel,
    mesh=plsc.VectorSubcoreMesh(core_axis_name="c", subcore_axis_name="s"),
    out_type=jax.ShapeDtypeStruct((BATCH * SEQ, 128), jnp.float32),
    scratch_types=[
        pltpu.VMEM((ROWS_PER_W,), jnp.int32),
        pltpu.VMEM((2 * CHUNK, DIM), jnp.float32),
        pltpu.VMEM((SEQ, DIM), jnp.float32),
        pltpu.SemaphoreType.DMA((2,)),
        pltpu.SemaphoreType.DMA((2,)),
    ],
    compiler_params=pltpu.CompilerParams(use_tc_tiling_on_sc=False),
)(_emb_body)


def kernel(inputs, word_table, pos_table):
    flat_idx = inputs.reshape(-1).astype(jnp.int32) * 4
    table_padded = jnp.pad(word_table, ((0, 0), (0, 128 - DIM))).reshape(
        4 * VOCAB, DIM
    )
    out = _emb(flat_idx, table_padded, pos_table)
    return out[:, :DIM].reshape(inputs.shape[0], inputs.shape[1], DIM)
